# baseline (device time: 83087 ns/iter reference)
import jax
import jax.numpy as jnp
from jax import lax
from jax.experimental import pallas as pl
from jax.experimental.pallas import tpu as pltpu

N_DEV = 4


def kernel(x, w_mat):
    m, _ = x.shape
    _, n = w_mat.shape

    def body(x_ref, w_ref, out_ref, comm_ref, send_sems, recv_sems):
        my = lax.axis_index("i")
        left = (my - 1) % N_DEV
        right = (my + 1) % N_DEV

        barrier_sem = pltpu.get_barrier_semaphore()
        for nbr in (left, right):
            pl.semaphore_signal(
                barrier_sem, inc=1,
                device_id=(nbr,), device_id_type=pl.DeviceIdType.MESH,
            )
        pl.semaphore_wait(barrier_sem, 2)

        partial = jnp.dot(x_ref[:, :], w_ref[:, :],
                          preferred_element_type=jnp.float32)
        out_ref[:, :] = partial
        comm_ref[0, :, :] = partial.astype(jnp.bfloat16)

        for h in range(N_DEV - 1):
            rdma = pltpu.make_async_remote_copy(
                src_ref=comm_ref.at[h],
                dst_ref=comm_ref.at[h + 1],
                send_sem=send_sems.at[h],
                recv_sem=recv_sems.at[h],
                device_id=(right,),
                device_id_type=pl.DeviceIdType.MESH,
            )
            rdma.start()
            rdma.wait()
            out_ref[:, :] = out_ref[:, :] + comm_ref[h + 1, :, :].astype(jnp.float32)

        y = out_ref[:, :]
        out_ref[:, :] = y * (1.0 / (1.0 + jnp.exp(-y)))

    return pl.pallas_call(
        body,
        out_shape=jax.ShapeDtypeStruct((m, n), jnp.float32),
        in_specs=[
            pl.BlockSpec(memory_space=pltpu.VMEM),
            pl.BlockSpec(memory_space=pltpu.VMEM),
        ],
        out_specs=pl.BlockSpec(memory_space=pltpu.VMEM),
        scratch_shapes=[
            pltpu.VMEM((N_DEV, m, n), jnp.bfloat16),
            pltpu.SemaphoreType.DMA((N_DEV - 1,)),
            pltpu.SemaphoreType.DMA((N_DEV - 1,)),
        ],
        compiler_params=pltpu.CompilerParams(collective_id=0),
    )(x, w_mat)


# device time: 32660 ns/iter; 2.5440x vs baseline; 2.5440x over previous
import jax
import jax.numpy as jnp
from jax import lax
from jax.experimental import pallas as pl
from jax.experimental.pallas import tpu as pltpu

N_DEV = 4


def kernel(x, w_mat):
    m, _ = x.shape
    _, n = w_mat.shape
    f32 = jnp.float32

    def body(x_ref, w_ref, out_ref,
             s1, r1, s2, r2, g1s, g1r, g2s, g2r,
             send_sems, recv_sems):
        my = lax.axis_index("i")
        pA = my + 1 - 2 * (my % 2)
        pB = 3 - my

        barrier_sem = pltpu.get_barrier_semaphore()
        for nbr in (pA, pB):
            pl.semaphore_signal(
                barrier_sem, inc=1,
                device_id=(nbr,), device_id_type=pl.DeviceIdType.MESH,
            )
        pl.semaphore_wait(barrier_sem, 2)

        out_ref[:, :] = jnp.dot(x_ref[:, :], w_ref[:, :],
                                preferred_element_type=f32)

        kA = jnp.where((my == 1) | (my == 2), 1, 0)
        kB = my // 2
        qA = my // 2
        qB = my % 2
        baseA = kA * 256
        baseB = 512 + kB * 256
        ownA = baseA + qA * 128
        ownB = baseB + qB * 128

        def exchange(p, sbuf, rbuf, size, a_off, b_off, tgtA, tgtB):
            sbuf[0, :, :] = out_ref[pl.ds(a_off, size), :].astype(jnp.bfloat16)
            sbuf[1, :, :] = out_ref[pl.ds(b_off, size), :].astype(jnp.bfloat16)
            rdmas = []
            for d, tgt in ((0, tgtA), (1, tgtB)):
                rdma = pltpu.make_async_remote_copy(
                    src_ref=sbuf.at[d],
                    dst_ref=rbuf.at[d],
                    send_sem=send_sems.at[p * 2 + d],
                    recv_sem=recv_sems.at[p * 2 + d],
                    device_id=(tgt,),
                    device_id_type=pl.DeviceIdType.MESH,
                )
                rdma.start()
                rdmas.append(rdma)
            for rdma in rdmas:
                rdma.wait()

        exchange(0, s1, r1, 256, (1 - kA) * 256, 512 + (1 - kB) * 256, pA, pB)
        out_ref[pl.ds(baseA, 256), :] = (
            out_ref[pl.ds(baseA, 256), :] + r1[0, :, :].astype(f32))
        out_ref[pl.ds(baseB, 256), :] = (
            out_ref[pl.ds(baseB, 256), :] + r1[1, :, :].astype(f32))

        exchange(1, s2, r2, 128,
                 baseA + (1 - qA) * 128, baseB + (1 - qB) * 128, pB, pA)
        out_ref[pl.ds(ownA, 128), :] = (
            out_ref[pl.ds(ownA, 128), :] + r2[0, :, :].astype(f32))
        out_ref[pl.ds(ownB, 128), :] = (
            out_ref[pl.ds(ownB, 128), :] + r2[1, :, :].astype(f32))

        for off in (ownA, ownB):
            y = out_ref[pl.ds(off, 128), :]
            out_ref[pl.ds(off, 128), :] = y * (1.0 / (1.0 + jnp.exp(-y)))

        exchange(2, g1s, g1r, 128, ownA, ownB, pB, pA)
        out_ref[pl.ds(baseA + (1 - qA) * 128, 128), :] = g1r[0, :, :].astype(f32)
        out_ref[pl.ds(baseB + (1 - qB) * 128, 128), :] = g1r[1, :, :].astype(f32)

        exchange(3, g2s, g2r, 256, baseA, baseB, pA, pB)
        out_ref[pl.ds((1 - kA) * 256, 256), :] = g2r[0, :, :].astype(f32)
        out_ref[pl.ds(512 + (1 - kB) * 256, 256), :] = g2r[1, :, :].astype(f32)

    bf16 = jnp.bfloat16
    return pl.pallas_call(
        body,
        out_shape=jax.ShapeDtypeStruct((m, n), f32),
        in_specs=[
            pl.BlockSpec(memory_space=pltpu.VMEM),
            pl.BlockSpec(memory_space=pltpu.VMEM),
        ],
        out_specs=pl.BlockSpec(memory_space=pltpu.VMEM),
        scratch_shapes=[
            pltpu.VMEM((2, 256, n), bf16),
            pltpu.VMEM((2, 256, n), bf16),
            pltpu.VMEM((2, 128, n), bf16),
            pltpu.VMEM((2, 128, n), bf16),
            pltpu.VMEM((2, 128, n), bf16),
            pltpu.VMEM((2, 128, n), bf16),
            pltpu.VMEM((2, 256, n), bf16),
            pltpu.VMEM((2, 256, n), bf16),
            pltpu.SemaphoreType.DMA((8,)),
            pltpu.SemaphoreType.DMA((8,)),
        ],
        compiler_params=pltpu.CompilerParams(collective_id=0),
    )(x, w_mat)


# device time: 27911 ns/iter; 2.9769x vs baseline; 1.1701x over previous
import jax
import jax.numpy as jnp
from jax import lax
from jax.experimental import pallas as pl
from jax.experimental.pallas import tpu as pltpu

N_DEV = 4
CHUNK = 2


def kernel(x, w_mat):
    m, _ = x.shape
    _, n = w_mat.shape
    f32 = jnp.float32
    ncol = n // CHUNK

    def body(x_ref, w_ref, out_ref,
             s1, r1, s2, r2, g1s, g1r, g2s, g2r,
             send_sems, recv_sems):
        my = lax.axis_index("i")
        pA = my + 1 - 2 * (my % 2)
        pB = 3 - my

        barrier_sem = pltpu.get_barrier_semaphore()
        for nbr in (pA, pB):
            pl.semaphore_signal(
                barrier_sem, inc=1,
                device_id=(nbr,), device_id_type=pl.DeviceIdType.MESH,
            )
        pl.semaphore_wait(barrier_sem, 2)

        kA = jnp.where((my == 1) | (my == 2), 1, 0)
        kB = my // 2
        qA = my // 2
        qB = my % 2
        baseA = kA * 256
        baseB = 512 + kB * 256
        ownA = baseA + qA * 128
        ownB = baseB + qB * 128

        phases = (
            (s1, r1, 256, (1 - kA) * 256, 512 + (1 - kB) * 256, pA, pB,
             baseA, baseB, "add"),
            (s2, r2, 128, baseA + (1 - qA) * 128, baseB + (1 - qB) * 128,
             pB, pA, ownA, ownB, "add"),
            (g1s, g1r, 128, ownA, ownB, pB, pA,
             baseA + (1 - qA) * 128, baseB + (1 - qB) * 128, "set"),
            (g2s, g2r, 256, baseA, baseB, pA, pB,
             (1 - kA) * 256, 512 + (1 - kB) * 256, "set"),
        )

        def stage_start(p, c):
            sbuf, rbuf, size, a_off, b_off, tgtA, tgtB = phases[p][:7]
            cs = slice(c * ncol, (c + 1) * ncol)
            rdmas = []
            for d, (off, tgt) in enumerate(((a_off, tgtA), (b_off, tgtB))):
                sbuf[d, :, cs] = out_ref[pl.ds(off, size), cs].astype(
                    jnp.bfloat16)
                rdma = pltpu.make_async_remote_copy(
                    src_ref=sbuf.at[d, :, cs],
                    dst_ref=rbuf.at[d, :, cs],
                    send_sem=send_sems.at[(p * 2 + d) * CHUNK + c],
                    recv_sem=recv_sems.at[(p * 2 + d) * CHUNK + c],
                    device_id=(tgt,),
                    device_id_type=pl.DeviceIdType.MESH,
                )
                rdma.start()
                rdmas.append(rdma)
            return rdmas

        def finish(p, c):
            rbuf, size = phases[p][1], phases[p][2]
            la, lb, mode = phases[p][7:]
            cs = slice(c * ncol, (c + 1) * ncol)
            for d, off in ((0, la), (1, lb)):
                recv = rbuf[d, :, cs].astype(f32)
                if mode == "add":
                    out_ref[pl.ds(off, size), cs] = (
                        out_ref[pl.ds(off, size), cs] + recv)
                else:
                    out_ref[pl.ds(off, size), cs] = recv
            if p == 1:
                for off in (ownA, ownB):
                    y = out_ref[pl.ds(off, 128), cs]
                    out_ref[pl.ds(off, 128), cs] = y * (
                        1.0 / (1.0 + jnp.exp(-y)))

        pending = {}
        for c in range(CHUNK):
            cs = slice(c * ncol, (c + 1) * ncol)
            out_ref[:, cs] = jnp.dot(x_ref[:, :], w_ref[:, cs],
                                     preferred_element_type=f32)
            pending[c] = stage_start(0, c)
        for p in range(1, 4):
            for c in range(CHUNK):
                for rdma in pending[c]:
                    rdma.wait()
                finish(p - 1, c)
                pending[c] = stage_start(p, c)
        for c in range(CHUNK):
            for rdma in pending[c]:
                rdma.wait()
            finish(3, c)

    bf16 = jnp.bfloat16
    return pl.pallas_call(
        body,
        out_shape=jax.ShapeDtypeStruct((m, n), f32),
        in_specs=[
            pl.BlockSpec(memory_space=pltpu.VMEM),
            pl.BlockSpec(memory_space=pltpu.VMEM),
        ],
        out_specs=pl.BlockSpec(memory_space=pltpu.VMEM),
        scratch_shapes=[
            pltpu.VMEM((2, 256, n), bf16),
            pltpu.VMEM((2, 256, n), bf16),
            pltpu.VMEM((2, 128, n), bf16),
            pltpu.VMEM((2, 128, n), bf16),
            pltpu.VMEM((2, 128, n), bf16),
            pltpu.VMEM((2, 128, n), bf16),
            pltpu.VMEM((2, 256, n), bf16),
            pltpu.VMEM((2, 256, n), bf16),
            pltpu.SemaphoreType.DMA((4 * 2 * CHUNK,)),
            pltpu.SemaphoreType.DMA((4 * 2 * CHUNK,)),
        ],
        compiler_params=pltpu.CompilerParams(collective_id=0),
    )(x, w_mat)


# device time: 27159 ns/iter; 3.0593x vs baseline; 1.0277x over previous
import jax
import jax.numpy as jnp
from jax import lax
from jax.experimental import pallas as pl
from jax.experimental.pallas import tpu as pltpu

N_DEV = 4
CHUNK = 4


def kernel(x, w_mat):
    m, _ = x.shape
    _, n = w_mat.shape
    f32 = jnp.float32
    ncol = n // CHUNK

    def body(x_ref, w_ref, out_ref,
             s1, r1, s2, r2, g1s, g1r, g2s, g2r,
             send_sems, recv_sems):
        my = lax.axis_index("i")
        pA = my + 1 - 2 * (my % 2)
        pB = 3 - my

        barrier_sem = pltpu.get_barrier_semaphore()
        for nbr in (pA, pB):
            pl.semaphore_signal(
                barrier_sem, inc=1,
                device_id=(nbr,), device_id_type=pl.DeviceIdType.MESH,
            )
        pl.semaphore_wait(barrier_sem, 2)

        kA = jnp.where((my == 1) | (my == 2), 1, 0)
        kB = my // 2
        qA = my // 2
        qB = my % 2
        baseA = kA * 256
        baseB = 512 + kB * 256
        ownA = baseA + qA * 128
        ownB = baseB + qB * 128

        phases = (
            (s1, r1, 256, (1 - kA) * 256, 512 + (1 - kB) * 256, pA, pB,
             baseA, baseB, "add"),
            (s2, r2, 128, baseA + (1 - qA) * 128, baseB + (1 - qB) * 128,
             pB, pA, ownA, ownB, "add"),
            (g1s, g1r, 128, ownA, ownB, pB, pA,
             baseA + (1 - qA) * 128, baseB + (1 - qB) * 128, "set"),
            (g2s, g2r, 256, baseA, baseB, pA, pB,
             (1 - kA) * 256, 512 + (1 - kB) * 256, "set"),
        )

        def stage_start(p, c):
            sbuf, rbuf, size, a_off, b_off, tgtA, tgtB = phases[p][:7]
            cs = slice(c * ncol, (c + 1) * ncol)
            rdmas = []
            for d, (off, tgt) in enumerate(((a_off, tgtA), (b_off, tgtB))):
                sbuf[d, :, cs] = out_ref[pl.ds(off, size), cs].astype(
                    jnp.bfloat16)
                rdma = pltpu.make_async_remote_copy(
                    src_ref=sbuf.at[d, :, cs],
                    dst_ref=rbuf.at[d, :, cs],
                    send_sem=send_sems.at[(p * 2 + d) * CHUNK + c],
                    recv_sem=recv_sems.at[(p * 2 + d) * CHUNK + c],
                    device_id=(tgt,),
                    device_id_type=pl.DeviceIdType.MESH,
                )
                rdma.start()
                rdmas.append(rdma)
            return rdmas

        def finish(p, c):
            rbuf, size = phases[p][1], phases[p][2]
            la, lb, mode = phases[p][7:]
            cs = slice(c * ncol, (c + 1) * ncol)
            for d, off in ((0, la), (1, lb)):
                recv = rbuf[d, :, cs].astype(f32)
                if mode == "add":
                    out_ref[pl.ds(off, size), cs] = (
                        out_ref[pl.ds(off, size), cs] + recv)
                else:
                    out_ref[pl.ds(off, size), cs] = recv
            if p == 1:
                for off in (ownA, ownB):
                    y = out_ref[pl.ds(off, 128), cs]
                    out_ref[pl.ds(off, 128), cs] = y * (
                        1.0 / (1.0 + jnp.exp(-y)))

        pending = {}
        for c in range(CHUNK):
            cs = slice(c * ncol, (c + 1) * ncol)
            out_ref[:, cs] = jnp.dot(x_ref[:, :], w_ref[:, cs],
                                     preferred_element_type=f32)
            pending[c] = stage_start(0, c)
        for p in range(1, 4):
            for c in range(CHUNK):
                for rdma in pending[c]:
                    rdma.wait()
                finish(p - 1, c)
                pending[c] = stage_start(p, c)
        for c in range(CHUNK):
            for rdma in pending[c]:
                rdma.wait()
            finish(3, c)

    bf16 = jnp.bfloat16
    return pl.pallas_call(
        body,
        out_shape=jax.ShapeDtypeStruct((m, n), f32),
        in_specs=[
            pl.BlockSpec(memory_space=pltpu.VMEM),
            pl.BlockSpec(memory_space=pltpu.VMEM),
        ],
        out_specs=pl.BlockSpec(memory_space=pltpu.VMEM),
        scratch_shapes=[
            pltpu.VMEM((2, 256, n), bf16),
            pltpu.VMEM((2, 256, n), bf16),
            pltpu.VMEM((2, 128, n), bf16),
            pltpu.VMEM((2, 128, n), bf16),
            pltpu.VMEM((2, 128, n), bf16),
            pltpu.VMEM((2, 128, n), bf16),
            pltpu.VMEM((2, 256, n), bf16),
            pltpu.VMEM((2, 256, n), bf16),
            pltpu.SemaphoreType.DMA((4 * 2 * CHUNK,)),
            pltpu.SemaphoreType.DMA((4 * 2 * CHUNK,)),
        ],
        compiler_params=pltpu.CompilerParams(collective_id=0),
    )(x, w_mat)


# device time: 26929 ns/iter; 3.0854x vs baseline; 1.0085x over previous
import jax
import jax.numpy as jnp
from jax import lax
from jax.experimental import pallas as pl
from jax.experimental.pallas import tpu as pltpu

N_DEV = 4
CHUNK = 4


def kernel(x, w_mat):
    m, _ = x.shape
    _, n = w_mat.shape
    f32 = jnp.float32
    bf16 = jnp.bfloat16
    ncol = n // CHUNK

    def body(x_ref, w_ref, out_ref,
             s1, r1, s2, r2, g2s, g2r, send_sems, recv_sems):
        my = lax.axis_index("i")
        pA = my + 1 - 2 * (my % 2)
        pB = 3 - my

        barrier_sem = pltpu.get_barrier_semaphore()
        for nbr in (pA, pB):
            pl.semaphore_signal(
                barrier_sem, inc=1,
                device_id=(nbr,), device_id_type=pl.DeviceIdType.MESH,
            )
        pl.semaphore_wait(barrier_sem, 2)

        kA = jnp.where((my == 1) | (my == 2), 1, 0)
        kB = my // 2
        qA = my // 2
        qB = my % 2
        baseA = kA * 256
        baseB = 512 + kB * 256

        dsets = (
            ((pA, pB, pB, pA), baseA, qA, (1 - kA) * 256),
            ((pB, pA, pA, pB), baseB, qB, 512 + (1 - kB) * 256),
        )

        def rdma(p, d, c, src, dst):
            op = pltpu.make_async_remote_copy(
                src_ref=src, dst_ref=dst,
                send_sem=send_sems.at[p, d, c],
                recv_sem=recv_sems.at[p, d, c],
                device_id=(dsets[d][0][p],),
                device_id_type=pl.DeviceIdType.MESH,
            )
            op.start()
            return op

        pending = {}
        for c in range(CHUNK):
            cs = slice(c * ncol, (c + 1) * ncol)
            for d, (_, base, q, sent0) in enumerate(dsets):
                s1[d, :, cs] = jnp.dot(
                    x_ref[pl.ds(sent0, 256), :], w_ref[:, cs],
                    preferred_element_type=f32).astype(bf16)
            ops = [rdma(0, d, c, s1.at[d, :, cs], r1.at[d, :, cs])
                   for d in range(2)]
            for d, (_, base, q, sent0) in enumerate(dsets):
                out_ref[pl.ds(base, 256), cs] = jnp.dot(
                    x_ref[pl.ds(base, 256), :], w_ref[:, cs],
                    preferred_element_type=f32)
            pending[c] = ops

        for c in range(CHUNK):
            cs = slice(c * ncol, (c + 1) * ncol)
            for op in pending[c]:
                op.wait()
            ops = []
            for d, (_, base, q, sent0) in enumerate(dsets):
                fq = (1 - q) * 128
                s2[d, :, cs] = (
                    out_ref[pl.ds(base + fq, 128), cs]
                    + r1[d, pl.ds(fq, 128), cs].astype(f32)).astype(bf16)
                ops.append(rdma(1, d, c, s2.at[d, :, cs], r2.at[d, :, cs]))
                own = base + q * 128
                out_ref[pl.ds(own, 128), cs] = (
                    out_ref[pl.ds(own, 128), cs]
                    + r1[d, pl.ds(q * 128, 128), cs].astype(f32))
            pending[c] = ops

        for c in range(CHUNK):
            cs = slice(c * ncol, (c + 1) * ncol)
            for op in pending[c]:
                op.wait()
            ops = []
            for d, (_, base, q, sent0) in enumerate(dsets):
                own = base + q * 128
                y = (out_ref[pl.ds(own, 128), cs]
                     + r2[d, :, cs].astype(f32))
                y = y * (1.0 / (1.0 + jnp.exp(-y)))
                out_ref[pl.ds(own, 128), cs] = y
                qs = pl.ds(q * 128, 128)
                g2s[d, qs, cs] = y.astype(bf16)
                ops.append(rdma(2, d, c, g2s.at[d, qs, cs],
                                g2s.at[d, qs, cs]))
            pending[c] = ops

        for c in range(CHUNK):
            cs = slice(c * ncol, (c + 1) * ncol)
            for op in pending[c]:
                op.wait()
            ops = []
            for d in range(2):
                ops.append(rdma(3, d, c, g2s.at[d, :, cs],
                                g2r.at[d, :, cs]))
            for d, (_, base, q, sent0) in enumerate(dsets):
                rq = pl.ds((1 - q) * 128, 128)
                out_ref[pl.ds(base + (1 - q) * 128, 128), cs] = (
                    g2s[d, rq, cs].astype(f32))
            pending[c] = ops

        for c in range(CHUNK):
            cs = slice(c * ncol, (c + 1) * ncol)
            for op in pending[c]:
                op.wait()
            out_ref[pl.ds((1 - kA) * 256, 256), cs] = (
                g2r[0, :, cs].astype(f32))
            out_ref[pl.ds(512 + (1 - kB) * 256, 256), cs] = (
                g2r[1, :, cs].astype(f32))

    return pl.pallas_call(
        body,
        out_shape=jax.ShapeDtypeStruct((m, n), f32),
        in_specs=[
            pl.BlockSpec(memory_space=pltpu.VMEM),
            pl.BlockSpec(memory_space=pltpu.VMEM),
        ],
        out_specs=pl.BlockSpec(memory_space=pltpu.VMEM),
        scratch_shapes=[
            pltpu.VMEM((2, 256, n), bf16),
            pltpu.VMEM((2, 256, n), bf16),
            pltpu.VMEM((2, 128, n), bf16),
            pltpu.VMEM((2, 128, n), bf16),
            pltpu.VMEM((2, 256, n), bf16),
            pltpu.VMEM((2, 256, n), bf16),
            pltpu.SemaphoreType.DMA((4, 2, CHUNK)),
            pltpu.SemaphoreType.DMA((4, 2, CHUNK)),
        ],
        compiler_params=pltpu.CompilerParams(collective_id=0),
    )(x, w_mat)


# device time: 26672 ns/iter; 3.1151x vs baseline; 1.0096x over previous
import jax
import jax.numpy as jnp
from jax import lax
from jax.experimental import pallas as pl
from jax.experimental.pallas import tpu as pltpu

N_DEV = 4
CHUNK = 4


def kernel(x, w_mat):
    m, _ = x.shape
    _, n = w_mat.shape
    f32 = jnp.float32
    bf16 = jnp.bfloat16
    ncol = n // CHUNK

    def body(x_ref, w_ref, out_ref,
             s1, r1, s2, r2, g2s, g2r, send_sems, recv_sems):
        my = lax.axis_index("i")
        pA = my + 1 - 2 * (my % 2)
        pB = 3 - my

        barrier_sem = pltpu.get_barrier_semaphore()
        for nbr in (pA, pB):
            pl.semaphore_signal(
                barrier_sem, inc=1,
                device_id=(nbr,), device_id_type=pl.DeviceIdType.MESH,
            )

        kA = jnp.where((my == 1) | (my == 2), 1, 0)
        kB = my // 2
        qA = my // 2
        qB = my % 2
        baseA = kA * 256
        baseB = 512 + kB * 256

        dsets = (
            ((pA, pB, pB, pA), baseA, qA, (1 - kA) * 256),
            ((pB, pA, pA, pB), baseB, qB, 512 + (1 - kB) * 256),
        )

        def rdma(p, d, c, src, dst):
            op = pltpu.make_async_remote_copy(
                src_ref=src, dst_ref=dst,
                send_sem=send_sems.at[p, d, c],
                recv_sem=recv_sems.at[p, d, c],
                device_id=(dsets[d][0][p],),
                device_id_type=pl.DeviceIdType.MESH,
            )
            op.start()
            return op

        pending = {}
        for c in range(CHUNK):
            cs = slice(c * ncol, (c + 1) * ncol)
            for d, (_, base, q, sent0) in enumerate(dsets):
                s1[d, :, cs] = jnp.dot(
                    x_ref[pl.ds(sent0, 256), :], w_ref[:, cs],
                    preferred_element_type=f32).astype(bf16)
            if c == 0:
                pl.semaphore_wait(barrier_sem, 2)
            ops = [rdma(0, d, c, s1.at[d, :, cs], r1.at[d, :, cs])
                   for d in range(2)]
            for d, (_, base, q, sent0) in enumerate(dsets):
                out_ref[pl.ds(base, 256), cs] = jnp.dot(
                    x_ref[pl.ds(base, 256), :], w_ref[:, cs],
                    preferred_element_type=f32)
            pending[c] = ops

        for c in range(CHUNK):
            cs = slice(c * ncol, (c + 1) * ncol)
            for op in pending[c]:
                op.wait()
            ops = []
            for d, (_, base, q, sent0) in enumerate(dsets):
                fq = (1 - q) * 128
                s2[d, :, cs] = (
                    out_ref[pl.ds(base + fq, 128), cs]
                    + r1[d, pl.ds(fq, 128), cs].astype(f32)).astype(bf16)
                ops.append(rdma(1, d, c, s2.at[d, :, cs], r2.at[d, :, cs]))
                own = base + q * 128
                out_ref[pl.ds(own, 128), cs] = (
                    out_ref[pl.ds(own, 128), cs]
                    + r1[d, pl.ds(q * 128, 128), cs].astype(f32))
            pending[c] = ops

        for c in range(CHUNK):
            cs = slice(c * ncol, (c + 1) * ncol)
            for op in pending[c]:
                op.wait()
            ops = []
            for d, (_, base, q, sent0) in enumerate(dsets):
                own = base + q * 128
                y = (out_ref[pl.ds(own, 128), cs]
                     + r2[d, :, cs].astype(f32))
                y = y * (1.0 / (1.0 + jnp.exp(-y)))
                out_ref[pl.ds(own, 128), cs] = y
                qs = pl.ds(q * 128, 128)
                g2s[d, qs, cs] = y.astype(bf16)
                ops.append(rdma(2, d, c, g2s.at[d, qs, cs],
                                g2s.at[d, qs, cs]))
            pending[c] = ops

        for c in range(CHUNK):
            cs = slice(c * ncol, (c + 1) * ncol)
            for op in pending[c]:
                op.wait()
            ops = []
            for d in range(2):
                ops.append(rdma(3, d, c, g2s.at[d, :, cs],
                                g2r.at[d, :, cs]))
            for d, (_, base, q, sent0) in enumerate(dsets):
                rq = pl.ds((1 - q) * 128, 128)
                out_ref[pl.ds(base + (1 - q) * 128, 128), cs] = (
                    g2s[d, rq, cs].astype(f32))
            pending[c] = ops

        for c in range(CHUNK):
            cs = slice(c * ncol, (c + 1) * ncol)
            for op in pending[c]:
                op.wait()
            out_ref[pl.ds((1 - kA) * 256, 256), cs] = (
                g2r[0, :, cs].astype(f32))
            out_ref[pl.ds(512 + (1 - kB) * 256, 256), cs] = (
                g2r[1, :, cs].astype(f32))

    return pl.pallas_call(
        body,
        out_shape=jax.ShapeDtypeStruct((m, n), f32),
        in_specs=[
            pl.BlockSpec(memory_space=pltpu.VMEM),
            pl.BlockSpec(memory_space=pltpu.VMEM),
        ],
        out_specs=pl.BlockSpec(memory_space=pltpu.VMEM),
        scratch_shapes=[
            pltpu.VMEM((2, 256, n), bf16),
            pltpu.VMEM((2, 256, n), bf16),
            pltpu.VMEM((2, 128, n), bf16),
            pltpu.VMEM((2, 128, n), bf16),
            pltpu.VMEM((2, 256, n), bf16),
            pltpu.VMEM((2, 256, n), bf16),
            pltpu.SemaphoreType.DMA((4, 2, CHUNK)),
            pltpu.SemaphoreType.DMA((4, 2, CHUNK)),
        ],
        compiler_params=pltpu.CompilerParams(collective_id=0),
    )(x, w_mat)
